# baseline (speedup 1.0000x reference)
import jax
import jax.numpy as jnp
from jax.experimental import pallas as pl
from jax.experimental.pallas import tpu as pltpu
from jax.experimental.pallas import tpu_sc as plsc


_D = 128
_MAXP = 90
_LANES = 16
_CH = 80
_NBUF = 4
_KPAD = 96
_BLK = 1024
_N_TC = 61440


def _sc_gather(idx3, embedding, n_sc, per_worker, nch, mesh):
    @pl.kernel(
        out_type=jax.ShapeDtypeStruct((n_sc, _D), jnp.float32),
        mesh=mesh,
        scratch_types=[
            pltpu.VMEM_SHARED((_MAXP, _D), jnp.float32),
            pltpu.VMEM((nch, _CH), jnp.int32),
            pltpu.VMEM((_NBUF, _CH, _D), jnp.float32),
            pltpu.SemaphoreType.DMA,
            pltpu.SemaphoreType.DMA((_NBUF,)),
            pltpu.SemaphoreType.DMA((_NBUF,)),
        ],
    )
    def gather_kernel(
        table_hbm, i_hbm, o_hbm, table_s, idx_v, out_v, isem, gsem, osem
    ):
        sid = jax.lax.axis_index("subcore")
        wid = sid * mesh.num_cores + jax.lax.axis_index("core")
        row_base = wid * per_worker

        pltpu.async_copy(i_hbm.at[wid], idx_v, isem)

        @pl.when(sid == 0)
        def _():
            pltpu.sync_copy(table_hbm, table_s)

        plsc.subcore_barrier()
        pltpu.make_async_copy(i_hbm.at[wid], idx_v, isem).wait()

        @pl.loop(0, nch)
        def _(c):
            for s in range(_CH // _LANES):
                sl = pl.ds(s * _LANES, _LANES)
                idx_v[c, sl] = jnp.minimum(
                    jnp.maximum(idx_v[c, sl], 0), _MAXP - 1
                )

        @pl.loop(0, nch, step=_NBUF)
        def _(c0):
            for u in range(_NBUF):
                c = c0 + u
                ob = out_v.at[u]

                @pl.when(c >= _NBUF)
                def _():
                    pltpu.make_async_copy(
                        ob,
                        o_hbm.at[pl.ds(row_base + (c - _NBUF) * _CH, _CH)],
                        osem.at[u],
                    ).wait()

                pltpu.async_copy(table_s.at[idx_v.at[c]], ob, gsem.at[u])

                up = (u - 1) % _NBUF

                @pl.when(c >= 1)
                def _():
                    pltpu.make_async_copy(
                        table_s.at[idx_v.at[c - 1]],
                        out_v.at[up],
                        gsem.at[up],
                    ).wait()
                    pltpu.async_copy(
                        out_v.at[up],
                        o_hbm.at[pl.ds(row_base + (c - 1) * _CH, _CH)],
                        osem.at[up],
                    )

        ul = (nch - 1) % _NBUF
        pltpu.make_async_copy(
            table_s.at[idx_v.at[nch - 1]], out_v.at[ul], gsem.at[ul]
        ).wait()
        pltpu.async_copy(
            out_v.at[ul],
            o_hbm.at[pl.ds(row_base + (nch - 1) * _CH, _CH)],
            osem.at[ul],
        )
        for u in range(_NBUF):
            cc = nch - _NBUF + u
            pltpu.make_async_copy(
                out_v.at[u],
                o_hbm.at[pl.ds(row_base + cc * _CH, _CH)],
                osem.at[u],
            ).wait()

    return gather_kernel(embedding, idx3)


def _tc_onehot_body(idx_ref, hi_ref, mid_ref, lo_ref, o_ref):
    idx = idx_ref[...]
    idx = jnp.minimum(jnp.maximum(idx, 0), _MAXP - 1)
    cols = jax.lax.broadcasted_iota(jnp.int32, (_BLK, _KPAD), 1)
    oh = (idx == cols).astype(jnp.bfloat16)
    acc = jnp.dot(oh, hi_ref[...], preferred_element_type=jnp.float32)
    acc = acc + jnp.dot(oh, mid_ref[...], preferred_element_type=jnp.float32)
    acc = acc + jnp.dot(oh, lo_ref[...], preferred_element_type=jnp.float32)
    o_ref[...] = acc


def _tc_gather(idx_tc, embedding):
    hi = embedding.astype(jnp.bfloat16)
    r1 = embedding - hi.astype(jnp.float32)
    mid = r1.astype(jnp.bfloat16)
    lo = (r1 - mid.astype(jnp.float32)).astype(jnp.bfloat16)
    pad = ((0, _KPAD - _MAXP), (0, 0))
    hi, mid, lo = (jnp.pad(x, pad) for x in (hi, mid, lo))

    n_tc = idx_tc.shape[0]
    idx2 = idx_tc.reshape(n_tc, 1)

    return pl.pallas_call(
        _tc_onehot_body,
        grid=(n_tc // _BLK,),
        in_specs=[
            pl.BlockSpec((_BLK, 1), lambda i: (i, 0)),
            pl.BlockSpec((_KPAD, _D), lambda i: (0, 0)),
            pl.BlockSpec((_KPAD, _D), lambda i: (0, 0)),
            pl.BlockSpec((_KPAD, _D), lambda i: (0, 0)),
        ],
        out_specs=pl.BlockSpec((_BLK, _D), lambda i: (i, 0)),
        out_shape=jax.ShapeDtypeStruct((n_tc, _D), jnp.float32),
    )(idx2, hi, mid, lo)


def kernel(cumulative_positions, embedding):
    b, t = cumulative_positions.shape
    n = b * t
    n_sc = n - _N_TC

    mesh = plsc.VectorSubcoreMesh(
        core_axis_name="core", subcore_axis_name="subcore"
    )
    num_workers = mesh.num_cores * mesh.num_subcores
    per_worker = n_sc // num_workers
    nch = per_worker // _CH

    idx_flat = cumulative_positions.reshape(n).astype(jnp.int32)
    idx3 = idx_flat[:n_sc].reshape(num_workers, nch, _CH)
    idx_tc = idx_flat[n_sc:]

    out_sc = _sc_gather(idx3, embedding, n_sc, per_worker, nch, mesh)
    out_tc = _tc_gather(idx_tc, embedding)
    return jnp.concatenate([out_sc, out_tc], axis=0).reshape(b, t, _D)


# speedup vs baseline: 1.5185x; 1.5185x over previous
import jax
import jax.numpy as jnp
from jax.experimental import pallas as pl
from jax.experimental.pallas import tpu as pltpu
from jax.experimental.pallas import tpu_sc as plsc


_D = 128
_MAXP = 90
_LANES = 16
_CH = 80
_NBUF = 4
_KPAD = 96
_BLK = 1024
_N_TC = 61440


def _sc_gather(idx3, embedding, n_out, per_worker, nch, mesh):
    @pl.kernel(
        out_type=jax.ShapeDtypeStruct((n_out, _D), jnp.float32),
        mesh=mesh,
        scratch_types=[
            pltpu.VMEM_SHARED((_MAXP, _D), jnp.float32),
            pltpu.VMEM((nch, _CH), jnp.int32),
            pltpu.VMEM((_NBUF, _CH, _D), jnp.float32),
            pltpu.SemaphoreType.DMA,
            pltpu.SemaphoreType.DMA((_NBUF,)),
            pltpu.SemaphoreType.DMA((_NBUF,)),
        ],
    )
    def gather_kernel(
        table_hbm, i_hbm, o_hbm, table_s, idx_v, out_v, isem, gsem, osem
    ):
        sid = jax.lax.axis_index("subcore")
        wid = sid * mesh.num_cores + jax.lax.axis_index("core")
        row_base = wid * per_worker

        pltpu.async_copy(i_hbm.at[wid], idx_v, isem)

        @pl.when(sid == 0)
        def _():
            pltpu.sync_copy(table_hbm, table_s)

        plsc.subcore_barrier()
        pltpu.make_async_copy(i_hbm.at[wid], idx_v, isem).wait()

        @pl.loop(0, nch)
        def _(c):
            for s in range(_CH // _LANES):
                sl = pl.ds(s * _LANES, _LANES)
                idx_v[c, sl] = jnp.minimum(
                    jnp.maximum(idx_v[c, sl], 0), _MAXP - 1
                )

        @pl.loop(0, nch, step=_NBUF)
        def _(c0):
            for u in range(_NBUF):
                c = c0 + u
                ob = out_v.at[u]

                @pl.when(c >= _NBUF)
                def _():
                    pltpu.make_async_copy(
                        ob,
                        o_hbm.at[pl.ds(row_base + (c - _NBUF) * _CH, _CH)],
                        osem.at[u],
                    ).wait()

                pltpu.async_copy(table_s.at[idx_v.at[c]], ob, gsem.at[u])

                up = (u - 1) % _NBUF

                @pl.when(c >= 1)
                def _():
                    pltpu.make_async_copy(
                        table_s.at[idx_v.at[c - 1]],
                        out_v.at[up],
                        gsem.at[up],
                    ).wait()
                    pltpu.async_copy(
                        out_v.at[up],
                        o_hbm.at[pl.ds(row_base + (c - 1) * _CH, _CH)],
                        osem.at[up],
                    )

        ul = (nch - 1) % _NBUF
        pltpu.make_async_copy(
            table_s.at[idx_v.at[nch - 1]], out_v.at[ul], gsem.at[ul]
        ).wait()
        pltpu.async_copy(
            out_v.at[ul],
            o_hbm.at[pl.ds(row_base + (nch - 1) * _CH, _CH)],
            osem.at[ul],
        )
        for u in range(_NBUF):
            cc = nch - _NBUF + u
            pltpu.make_async_copy(
                out_v.at[u],
                o_hbm.at[pl.ds(row_base + cc * _CH, _CH)],
                osem.at[u],
            ).wait()

    return gather_kernel(embedding, idx3)


def _tc_onehot_body(idx_ref, hi_ref, mid_ref, lo_ref, prev_ref, o_ref):
    del prev_ref
    idx = idx_ref[...]
    idx = jnp.minimum(jnp.maximum(idx, 0), _MAXP - 1)
    cols = jax.lax.broadcasted_iota(jnp.int32, (_BLK, _KPAD), 1)
    oh = (idx == cols).astype(jnp.bfloat16)
    acc = jnp.dot(oh, hi_ref[...], preferred_element_type=jnp.float32)
    acc = acc + jnp.dot(oh, mid_ref[...], preferred_element_type=jnp.float32)
    acc = acc + jnp.dot(oh, lo_ref[...], preferred_element_type=jnp.float32)
    o_ref[...] = acc


def _tc_gather(idx_tc, embedding, prev, n_sc):
    hi = embedding.astype(jnp.bfloat16)
    r1 = embedding - hi.astype(jnp.float32)
    mid = r1.astype(jnp.bfloat16)
    lo = (r1 - mid.astype(jnp.float32)).astype(jnp.bfloat16)
    pad = ((0, _KPAD - _MAXP), (0, 0))
    hi, mid, lo = (jnp.pad(x, pad) for x in (hi, mid, lo))

    n_tc = idx_tc.shape[0]
    n = prev.shape[0]
    base = n_sc // _BLK
    idx2 = idx_tc.reshape(n_tc, 1)

    return pl.pallas_call(
        _tc_onehot_body,
        grid=(n_tc // _BLK,),
        in_specs=[
            pl.BlockSpec((_BLK, 1), lambda i: (i, 0)),
            pl.BlockSpec((_KPAD, _D), lambda i: (0, 0)),
            pl.BlockSpec((_KPAD, _D), lambda i: (0, 0)),
            pl.BlockSpec((_KPAD, _D), lambda i: (0, 0)),
            pl.BlockSpec((8, _D), lambda i: (0, 0)),
        ],
        out_specs=pl.BlockSpec((_BLK, _D), lambda i: (base + i, 0)),
        out_shape=jax.ShapeDtypeStruct((n, _D), jnp.float32),
        input_output_aliases={4: 0},
    )(idx2, hi, mid, lo, prev)


def kernel(cumulative_positions, embedding):
    b, t = cumulative_positions.shape
    n = b * t
    n_sc = n - _N_TC

    mesh = plsc.VectorSubcoreMesh(
        core_axis_name="core", subcore_axis_name="subcore"
    )
    num_workers = mesh.num_cores * mesh.num_subcores
    per_worker = n_sc // num_workers
    nch = per_worker // _CH

    idx_flat = cumulative_positions.reshape(n).astype(jnp.int32)
    idx3 = idx_flat[:n_sc].reshape(num_workers, nch, _CH)
    idx_tc = idx_flat[n_sc:]

    out_sc = _sc_gather(idx3, embedding, n, per_worker, nch, mesh)
    out = _tc_gather(idx_tc, embedding, out_sc, n_sc)
    return out.reshape(b, t, _D)


# speedup vs baseline: 2.4668x; 1.6246x over previous
import jax
import jax.numpy as jnp
from jax.experimental import pallas as pl
from jax.experimental.pallas import tpu as pltpu
from jax.experimental.pallas import tpu_sc as plsc


_D = 128
_MAXP = 90
_LANES = 16
_CH = 80
_NBUF = 8


def kernel(cumulative_positions, embedding):
    b, t = cumulative_positions.shape
    n = b * t

    mesh = plsc.VectorSubcoreMesh(
        core_axis_name="core", subcore_axis_name="subcore"
    )
    num_workers = mesh.num_cores * mesh.num_subcores
    per_worker = n // num_workers
    nch = per_worker // _CH

    idx3 = cumulative_positions.reshape(num_workers, nch, _CH).astype(
        jnp.int32
    )

    @pl.kernel(
        out_type=jax.ShapeDtypeStruct((n, _D), jnp.float32),
        mesh=mesh,
        scratch_types=[
            pltpu.VMEM_SHARED((_MAXP, _D), jnp.float32),
            pltpu.VMEM((nch, _CH), jnp.int32),
            pltpu.VMEM((_NBUF, _CH, _D), jnp.float32),
            pltpu.SemaphoreType.DMA,
            pltpu.SemaphoreType.DMA((_NBUF,)),
            pltpu.SemaphoreType.DMA((_NBUF,)),
        ],
    )
    def gather_kernel(
        table_hbm, i_hbm, o_hbm, table_s, idx_v, out_v, isem, gsem, osem
    ):
        sid = jax.lax.axis_index("subcore")
        wid = sid * mesh.num_cores + jax.lax.axis_index("core")
        row_base = wid * per_worker

        pltpu.async_copy(i_hbm.at[wid], idx_v, isem)

        @pl.when(sid == 0)
        def _():
            pltpu.sync_copy(table_hbm, table_s)

        plsc.subcore_barrier()
        pltpu.make_async_copy(i_hbm.at[wid], idx_v, isem).wait()

        @pl.loop(0, nch)
        def _(c):
            for s in range(_CH // _LANES):
                sl = pl.ds(s * _LANES, _LANES)
                idx_v[c, sl] = jnp.minimum(
                    jnp.maximum(idx_v[c, sl], 0), _MAXP - 1
                )

        @pl.loop(0, nch, step=_NBUF)
        def _(c0):
            for u in range(_NBUF):
                c = c0 + u
                ob = out_v.at[u]

                @pl.when(c >= _NBUF)
                def _():
                    pltpu.make_async_copy(
                        ob,
                        o_hbm.at[pl.ds(row_base + (c - _NBUF) * _CH, _CH)],
                        osem.at[u],
                    ).wait()

                pltpu.async_copy(table_s.at[idx_v.at[c]], ob, gsem.at[u])

                up = (u - 1) % _NBUF

                @pl.when(c >= 1)
                def _():
                    pltpu.make_async_copy(
                        table_s.at[idx_v.at[c - 1]],
                        out_v.at[up],
                        gsem.at[up],
                    ).wait()
                    pltpu.async_copy(
                        out_v.at[up],
                        o_hbm.at[pl.ds(row_base + (c - 1) * _CH, _CH)],
                        osem.at[up],
                    )

        ul = (nch - 1) % _NBUF
        pltpu.make_async_copy(
            table_s.at[idx_v.at[nch - 1]], out_v.at[ul], gsem.at[ul]
        ).wait()
        pltpu.async_copy(
            out_v.at[ul],
            o_hbm.at[pl.ds(row_base + (nch - 1) * _CH, _CH)],
            osem.at[ul],
        )
        for u in range(_NBUF):
            cc = nch - _NBUF + u
            pltpu.make_async_copy(
                out_v.at[u],
                o_hbm.at[pl.ds(row_base + cc * _CH, _CH)],
                osem.at[u],
            ).wait()

    out = gather_kernel(embedding, idx3)
    return out.reshape(b, t, _D)


# speedup vs baseline: 2.4685x; 1.0007x over previous
import jax
import jax.numpy as jnp
from jax.experimental import pallas as pl
from jax.experimental.pallas import tpu as pltpu
from jax.experimental.pallas import tpu_sc as plsc


_D = 128
_MAXP = 90
_LANES = 16
_CH = 80
_NBUF = 4


def kernel(cumulative_positions, embedding):
    b, t = cumulative_positions.shape
    n = b * t

    mesh = plsc.VectorSubcoreMesh(
        core_axis_name="core", subcore_axis_name="subcore"
    )
    num_workers = mesh.num_cores * mesh.num_subcores
    per_worker = n // num_workers
    nch = per_worker // _CH

    idx3 = cumulative_positions.reshape(num_workers, nch, _CH).astype(
        jnp.int32
    )

    @pl.kernel(
        out_type=jax.ShapeDtypeStruct((n, _D), jnp.float32),
        mesh=mesh,
        scratch_types=[
            pltpu.VMEM_SHARED((_MAXP, _D), jnp.float32),
            pltpu.VMEM((nch, _CH), jnp.int32),
            pltpu.VMEM((_NBUF, _CH, _D), jnp.float32),
            pltpu.SemaphoreType.DMA,
            pltpu.SemaphoreType.DMA((_NBUF,)),
            pltpu.SemaphoreType.DMA((_NBUF,)),
        ],
    )
    def gather_kernel(
        table_hbm, i_hbm, o_hbm, table_s, idx_v, out_v, isem, gsem, osem
    ):
        sid = jax.lax.axis_index("subcore")
        wid = sid * mesh.num_cores + jax.lax.axis_index("core")
        row_base = wid * per_worker

        pltpu.async_copy(i_hbm.at[wid], idx_v, isem)

        @pl.when(sid == 0)
        def _():
            pltpu.sync_copy(table_hbm, table_s)

        plsc.subcore_barrier()
        pltpu.make_async_copy(i_hbm.at[wid], idx_v, isem).wait()

        @pl.loop(0, nch)
        def _(c):
            for s in range(_CH // _LANES):
                sl = pl.ds(s * _LANES, _LANES)
                idx_v[c, sl] = jnp.minimum(
                    jnp.maximum(idx_v[c, sl], 0), _MAXP - 1
                )

        @pl.loop(0, nch, step=_NBUF)
        def _(c0):
            for u in range(_NBUF):
                c = c0 + u
                ob = out_v.at[u]

                @pl.when(c >= _NBUF)
                def _():
                    pltpu.make_async_copy(
                        ob,
                        o_hbm.at[pl.ds(row_base + (c - _NBUF) * _CH, _CH)],
                        osem.at[u],
                    ).wait()

                pltpu.async_copy(table_s.at[idx_v.at[c]], ob, gsem.at[u])

                up = (u - 1) % _NBUF

                @pl.when(c >= 1)
                def _():
                    pltpu.make_async_copy(
                        table_s.at[idx_v.at[c - 1]],
                        out_v.at[up],
                        gsem.at[up],
                    ).wait()
                    pltpu.async_copy(
                        out_v.at[up],
                        o_hbm.at[pl.ds(row_base + (c - 1) * _CH, _CH)],
                        osem.at[up],
                    )

        ul = (nch - 1) % _NBUF
        pltpu.make_async_copy(
            table_s.at[idx_v.at[nch - 1]], out_v.at[ul], gsem.at[ul]
        ).wait()
        pltpu.async_copy(
            out_v.at[ul],
            o_hbm.at[pl.ds(row_base + (nch - 1) * _CH, _CH)],
            osem.at[ul],
        )
        for u in range(_NBUF):
            cc = nch - _NBUF + u
            pltpu.make_async_copy(
                out_v.at[u],
                o_hbm.at[pl.ds(row_base + cc * _CH, _CH)],
                osem.at[u],
            ).wait()

    out = gather_kernel(embedding, idx3)
    return out.reshape(b, t, _D)
